# baseline (speedup 1.0000x reference)
import functools

import jax
import jax.numpy as jnp
from jax import lax
from jax.experimental import pallas as pl
from jax.experimental.pallas import tpu as pltpu

BN_EPS = 1e-5
LEAKY_SLOPE = 0.3
CPAD = 16
PAD = 128
IMGB = 8
_VMEM_LIMIT = 64 * 1024 * 1024


def _tree_sum(chunks):
    while len(chunks) > 1:
        nxt = [chunks[i] + chunks[i + 1] for i in range(0, len(chunks) - 1, 2)]
        if len(chunks) % 2:
            nxt.append(chunks[-1])
        chunks = nxt
    return chunks[0]


def _chunk_stats(v, nchunk):
    cs = [v[:, k * 128:(k + 1) * 128] for k in range(nchunk)]
    s = _tree_sum(cs)
    q = _tree_sum([c * c for c in cs])
    return s, q


STATB = 4


def _stats_kernel(x_ref, xb_ref, acc_ref, *, nchunk):
    i = pl.program_id(0)

    @pl.when(i == 0)
    def _():
        acc_ref[...] = jnp.zeros_like(acc_ref)

    B, C = x_ref.shape[0], x_ref.shape[1]
    HW = x_ref.shape[2] * x_ref.shape[3]
    flats = []
    for b in range(B):
        f = x_ref[b].reshape(C, HW)
        xb_ref[b * C:(b + 1) * C, :] = f.astype(jnp.bfloat16)
        flats.append(f)
    x = _tree_sum(flats)
    xx = _tree_sum([f * f for f in flats])
    s = _tree_sum([x[:, k * 128:(k + 1) * 128] for k in range(nchunk)])
    q = _tree_sum([xx[:, k * 128:(k + 1) * 128] for k in range(nchunk)])
    acc_ref[0:C, :] += s
    acc_ref[CPAD:CPAD + C, :] += q


def _channel_stats(x_nchw):
    N, C, H, W = x_nchw.shape
    HW = H * W
    xb, acc = pl.pallas_call(
        functools.partial(_stats_kernel, nchunk=HW // 128),
        grid=(N // STATB,),
        in_specs=[pl.BlockSpec((STATB, C, H, W), lambda i: (i, 0, 0, 0))],
        out_specs=[
            pl.BlockSpec((STATB * C, HW), lambda i: (i, 0)),
            pl.BlockSpec((2 * CPAD, 128), lambda i: (0, 0)),
        ],
        out_shape=[
            jax.ShapeDtypeStruct((N * C, HW), jnp.bfloat16),
            jax.ShapeDtypeStruct((2 * CPAD, 128), jnp.float32),
        ],
        compiler_params=pltpu.CompilerParams(
            dimension_semantics=("arbitrary",),
            vmem_limit_bytes=_VMEM_LIMIT),
    )(x_nchw)
    count = float(N * HW)
    s = jnp.sum(acc[0:C, :], axis=1)
    ss = jnp.sum(acc[CPAD:CPAD + C, :], axis=1)
    mean = s / count
    var = jnp.maximum(ss / count - mean * mean, 0.0)
    inv_std = 1.0 / jnp.sqrt(var + BN_EPS)
    return xb, mean, inv_std


def _make_aux(mean, inv_std, gamma, beta, group, c, nimg):
    scale = gamma.reshape(c) * inv_std
    shift = beta.reshape(c) - mean * scale
    sg = jnp.zeros((group,), jnp.float32).at[0:c].set(scale)
    hg = jnp.zeros((group,), jnp.float32).at[0:c].set(shift)
    aux = jnp.zeros((nimg * group, 8), jnp.float32)
    aux = aux.at[:, 0].set(jnp.tile(sg, nimg))
    return aux.at[:, 1].set(jnp.tile(hg, nimg))


def _retile_taps(wt, c):
    taps = wt[:, :9 * c].reshape(CPAD, 9, c)
    taps = jnp.pad(taps, ((0, 0), (0, 0), (0, CPAD - c)))
    return taps.reshape(CPAD, 9 * CPAD)


def _bias_band(bias):
    fo = bias.shape[1]
    return jnp.zeros((CPAD, 8), jnp.float32).at[:fo, 0].set(bias[0])


def _leaky_bn(x, aux_ref):
    a = x * aux_ref[:, 0:1] + aux_ref[:, 1:2]
    return jnp.maximum(a, LEAKY_SLOPE * a)


def _col_masks(C, W, HW):
    col = lax.broadcasted_iota(jnp.int32, (C, HW), 1) % W
    return col > 0, col < (W - 1)


def _fill_taps(apad_ref, patch_ref, b, group, C, W, HW, masks):
    mask_l, mask_r = masks
    t = 0
    for dy in range(3):
        for dx in range(3):
            off = (dy - 1) * W + (dx - 1)
            s0 = b * group
            src = apad_ref[s0:s0 + C, PAD + off:PAD + off + HW]
            if dx == 0:
                src = jnp.where(mask_l, src, jnp.zeros_like(src))
            elif dx == 2:
                src = jnp.where(mask_r, src, jnp.zeros_like(src))
            patch_ref[t * CPAD:t * CPAD + C, b * HW:(b + 1) * HW] = src
            t += 1


def _conv0_kernel(aux_ref, w_ref, x_ref, h_ref, acc_ref, apad_ref, patch_ref,
                  *, C, W, HW):
    i = pl.program_id(0)

    @pl.when(i == 0)
    def _():
        acc_ref[...] = jnp.zeros_like(acc_ref)
        apad_ref[...] = jnp.zeros_like(apad_ref)
        patch_ref[...] = jnp.zeros_like(patch_ref)
        ones = patch_ref[9 * CPAD:, :]
        patch_ref[9 * CPAD:, :] = jnp.ones_like(ones)

    a = _leaky_bn(x_ref[...], aux_ref).astype(jnp.bfloat16)
    apad_ref[:, PAD:PAD + HW] = a
    masks = _col_masks(C, W, HW)
    for b in range(IMGB):
        _fill_taps(apad_ref, patch_ref, b, C, C, W, HW, masks)
    h = jnp.dot(w_ref[...], patch_ref[...],
                preferred_element_type=jnp.float32)
    for b in range(IMGB):
        h_ref[b * CPAD:(b + 1) * CPAD, :] = (
            h[:, b * HW:(b + 1) * HW].astype(jnp.bfloat16))
    s, q = _chunk_stats(h, IMGB * (HW // 128))
    acc_ref[0:CPAD, :] += s
    acc_ref[CPAD:, :] += q


def _conv0(xb, aux0, w0b, C, W, HW):
    R = xb.shape[0]
    N = R // C
    rows = IMGB * C
    h, acc = pl.pallas_call(
        functools.partial(_conv0_kernel, C=C, W=W, HW=HW),
        grid=(R // rows,),
        in_specs=[
            pl.BlockSpec(aux0.shape, lambda i: (0, 0)),
            pl.BlockSpec(w0b.shape, lambda i: (0, 0)),
            pl.BlockSpec((rows, HW), lambda i: (i, 0)),
        ],
        out_specs=[
            pl.BlockSpec((IMGB * CPAD, HW), lambda i: (i, 0)),
            pl.BlockSpec((2 * CPAD, 128), lambda i: (0, 0)),
        ],
        out_shape=[
            jax.ShapeDtypeStruct((N * CPAD, HW), jnp.bfloat16),
            jax.ShapeDtypeStruct((2 * CPAD, 128), jnp.float32),
        ],
        scratch_shapes=[
            pltpu.VMEM((rows, HW + 2 * PAD), jnp.bfloat16),
            pltpu.VMEM((9 * CPAD + 8, IMGB * HW), jnp.bfloat16),
        ],
        compiler_params=pltpu.CompilerParams(
            dimension_semantics=("arbitrary",),
            vmem_limit_bytes=_VMEM_LIMIT),
    )(aux0, w0b, xb)
    return h, acc


IMGB1 = 8


def _conv1_kernel(aux_ref, w_ref, h_ref, x_ref, out_ref, apad_ref, patch_ref,
                  *, C, W, HW):
    i = pl.program_id(0)
    fout, H = out_ref.shape[1], out_ref.shape[2]

    @pl.when(i == 0)
    def _():
        apad_ref[...] = jnp.zeros_like(apad_ref)
        patch_ref[...] = jnp.zeros_like(patch_ref)
        ones = patch_ref[10 * CPAD:, :]
        patch_ref[10 * CPAD:, :] = jnp.ones_like(ones)

    a = _leaky_bn(h_ref[...].astype(jnp.float32), aux_ref).astype(jnp.bfloat16)
    apad_ref[:, PAD:PAD + HW] = a
    masks = _col_masks(C, W, HW)
    for b in range(IMGB1):
        _fill_taps(apad_ref, patch_ref, b, CPAD, C, W, HW, masks)
        patch_ref[9 * CPAD:9 * CPAD + C, b * HW:(b + 1) * HW] = (
            x_ref[b * C:(b + 1) * C, :])
    out = jnp.dot(w_ref[...], patch_ref[...],
                  preferred_element_type=jnp.float32)
    for b in range(IMGB1):
        out_ref[b] = out[0:fout, b * HW:(b + 1) * HW].reshape(fout, H, W)


def _conv1(h2, xb, aux1, w1b, C, fout, H, W):
    HW = H * W
    R = xb.shape[0]
    N = R // C
    rows = IMGB1 * C
    return pl.pallas_call(
        functools.partial(_conv1_kernel, C=C, W=W, HW=HW),
        grid=(R // rows,),
        in_specs=[
            pl.BlockSpec(aux1.shape, lambda i: (0, 0)),
            pl.BlockSpec(w1b.shape, lambda i: (0, 0)),
            pl.BlockSpec((IMGB1 * CPAD, HW), lambda i: (i, 0)),
            pl.BlockSpec((rows, HW), lambda i: (i, 0)),
        ],
        out_specs=pl.BlockSpec((IMGB1, fout, H, W), lambda i: (i, 0, 0, 0)),
        out_shape=jax.ShapeDtypeStruct((N, fout, H, W), jnp.float32),
        scratch_shapes=[
            pltpu.VMEM((IMGB1 * CPAD, HW + 2 * PAD), jnp.bfloat16),
            pltpu.VMEM((10 * CPAD + 8, IMGB1 * HW), jnp.bfloat16),
        ],
        compiler_params=pltpu.CompilerParams(
            dimension_semantics=("arbitrary",),
            vmem_limit_bytes=_VMEM_LIMIT),
    )(aux1, w1b, h2, xb)


def kernel(x_nchw, w0, b0, w1, b1, bn0_g, bn0_b, bn1_g, bn1_b):
    N, C, H, W = x_nchw.shape
    HW = H * W
    fout = b1.shape[1]

    w0t = jnp.transpose(w0)[0:CPAD, :]
    w1t = jnp.transpose(w1)[0:CPAD, :]
    w0b = jnp.concatenate(
        [_retile_taps(w0t, C), _bias_band(b0)],
        axis=1).astype(jnp.bfloat16)
    w1b = jnp.concatenate(
        [_retile_taps(w1t, C), w1t[:, 9 * C:9 * C + C],
         jnp.zeros((CPAD, CPAD - C), jnp.float32), _bias_band(b1)],
        axis=1).astype(jnp.bfloat16)

    xb, mean0, inv_std0 = _channel_stats(x_nchw)
    aux0 = _make_aux(mean0, inv_std0, bn0_g, bn0_b, C, C, IMGB)

    h2, acc1 = _conv0(xb, aux0, w0b, C, W, HW)

    count = float(N * HW)
    s1 = jnp.sum(acc1[0:C, :], axis=1)
    ss1 = jnp.sum(acc1[CPAD:CPAD + C, :], axis=1)
    mean1 = s1 / count
    var1 = jnp.maximum(ss1 / count - mean1 * mean1, 0.0)
    inv_std1 = 1.0 / jnp.sqrt(var1 + BN_EPS)
    aux1 = _make_aux(mean1, inv_std1, bn1_g, bn1_b, CPAD, C, IMGB1)

    return _conv1(h2, xb, aux1, w1b, C, fout, H, W)


# speedup vs baseline: 1.2678x; 1.2678x over previous
import functools

import jax
import jax.numpy as jnp
from jax import lax
from jax.experimental import pallas as pl
from jax.experimental.pallas import tpu as pltpu

BN_EPS = 1e-5
LEAKY_SLOPE = 0.3
CPAD = 16
PAD = 128
IMGB = 8
_VMEM_LIMIT = 64 * 1024 * 1024


def _tree_sum(chunks):
    while len(chunks) > 1:
        nxt = [chunks[i] + chunks[i + 1] for i in range(0, len(chunks) - 1, 2)]
        if len(chunks) % 2:
            nxt.append(chunks[-1])
        chunks = nxt
    return chunks[0]


def _chunk_stats(v, nchunk):
    cs = [v[:, k * 128:(k + 1) * 128] for k in range(nchunk)]
    s = _tree_sum(cs)
    q = _tree_sum([c * c for c in cs])
    return s, q


STATB = 4


def _stats_kernel(x_ref, xb_ref, acc_ref, *, nchunk):
    i = pl.program_id(0)

    @pl.when(i == 0)
    def _():
        acc_ref[...] = jnp.zeros_like(acc_ref)

    B, C = x_ref.shape[0], x_ref.shape[1]
    HW = x_ref.shape[2] * x_ref.shape[3]
    flats = []
    for b in range(B):
        f = x_ref[b].reshape(C, HW)
        xb_ref[b * C:(b + 1) * C, :] = f.astype(jnp.bfloat16)
        flats.append(f)
    x = _tree_sum(flats)
    xx = _tree_sum([f * f for f in flats])
    s = _tree_sum([x[:, k * 128:(k + 1) * 128] for k in range(nchunk)])
    q = _tree_sum([xx[:, k * 128:(k + 1) * 128] for k in range(nchunk)])
    acc_ref[0:C, :] += s
    acc_ref[CPAD:CPAD + C, :] += q


def _channel_stats(x32):
    N, C, H, W = x32.shape
    HW = H * W
    xb, acc = pl.pallas_call(
        functools.partial(_stats_kernel, nchunk=HW // 128),
        grid=(N // STATB,),
        in_specs=[pl.BlockSpec((STATB, C, H, W), lambda i: (i, 0, 0, 0))],
        out_specs=[
            pl.BlockSpec((STATB * C, HW), lambda i: (i, 0)),
            pl.BlockSpec((2 * CPAD, 128), lambda i: (0, 0)),
        ],
        out_shape=[
            jax.ShapeDtypeStruct((N * C, HW), jnp.bfloat16),
            jax.ShapeDtypeStruct((2 * CPAD, 128), jnp.float32),
        ],
        compiler_params=pltpu.CompilerParams(
            dimension_semantics=("arbitrary",),
            vmem_limit_bytes=_VMEM_LIMIT),
    )(x32)
    count = float(N * HW)
    s = jnp.sum(acc[0:C, :], axis=1)
    ss = jnp.sum(acc[CPAD:CPAD + C, :], axis=1)
    mean = s / count
    var = jnp.maximum(ss / count - mean * mean, 0.0)
    inv_std = 1.0 / jnp.sqrt(var + BN_EPS)
    return xb, mean, inv_std


def _make_aux(mean, inv_std, gamma, beta, group, c, nimg):
    scale = gamma.reshape(c) * inv_std
    shift = beta.reshape(c) - mean * scale
    sg = jnp.zeros((group,), jnp.float32).at[0:c].set(scale)
    hg = jnp.zeros((group,), jnp.float32).at[0:c].set(shift)
    aux = jnp.zeros((nimg * group, 8), jnp.float32)
    aux = aux.at[:, 0].set(jnp.tile(sg, nimg))
    return aux.at[:, 1].set(jnp.tile(hg, nimg))


def _retile_taps(wt, c):
    taps = wt[:, :9 * c].reshape(CPAD, 9, c)
    taps = jnp.pad(taps, ((0, 0), (0, 0), (0, CPAD - c)))
    return taps.reshape(CPAD, 9 * CPAD)


def _bias_band(bias):
    fo = bias.shape[1]
    return jnp.zeros((CPAD, 8), jnp.float32).at[:fo, 0].set(bias[0])


def _leaky_bn(x, aux_ref):
    a = x * aux_ref[:, 0:1] + aux_ref[:, 1:2]
    return jnp.maximum(a, LEAKY_SLOPE * a)


def _col_masks(C, W, HW):
    col = lax.broadcasted_iota(jnp.int32, (C, HW), 1) % W
    return col > 0, col < (W - 1)


def _fill_taps(apad_ref, patch_ref, b, group, C, W, HW, masks):
    mask_l, mask_r = masks
    t = 0
    for dy in range(3):
        for dx in range(3):
            off = (dy - 1) * W + (dx - 1)
            s0 = b * group
            src = apad_ref[s0:s0 + C, PAD + off:PAD + off + HW]
            if dx == 0:
                src = jnp.where(mask_l, src, jnp.zeros_like(src))
            elif dx == 2:
                src = jnp.where(mask_r, src, jnp.zeros_like(src))
            patch_ref[t * CPAD:t * CPAD + C, b * HW:(b + 1) * HW] = src
            t += 1


def _conv0_kernel(aux_ref, w_ref, x_ref, h_ref, acc_ref, apad_ref, patch_ref,
                  *, C, W, HW):
    i = pl.program_id(0)

    @pl.when(i == 0)
    def _():
        acc_ref[...] = jnp.zeros_like(acc_ref)
        apad_ref[...] = jnp.zeros_like(apad_ref)
        patch_ref[...] = jnp.zeros_like(patch_ref)
        ones = patch_ref[9 * CPAD:, :]
        patch_ref[9 * CPAD:, :] = jnp.ones_like(ones)

    a = _leaky_bn(x_ref[...], aux_ref).astype(jnp.bfloat16)
    apad_ref[:, PAD:PAD + HW] = a
    masks = _col_masks(C, W, HW)
    for b in range(IMGB):
        _fill_taps(apad_ref, patch_ref, b, C, C, W, HW, masks)
    h = jnp.dot(w_ref[...], patch_ref[...],
                preferred_element_type=jnp.float32)
    for b in range(IMGB):
        h_ref[b * CPAD:(b + 1) * CPAD, :] = (
            h[:, b * HW:(b + 1) * HW].astype(jnp.bfloat16))
    s, q = _chunk_stats(h, IMGB * (HW // 128))
    acc_ref[0:CPAD, :] += s
    acc_ref[CPAD:, :] += q


def _conv0(xb, aux0, w0b, C, W, HW):
    R = xb.shape[0]
    N = R // C
    rows = IMGB * C
    h, acc = pl.pallas_call(
        functools.partial(_conv0_kernel, C=C, W=W, HW=HW),
        grid=(R // rows,),
        in_specs=[
            pl.BlockSpec(aux0.shape, lambda i: (0, 0)),
            pl.BlockSpec(w0b.shape, lambda i: (0, 0)),
            pl.BlockSpec((rows, HW), lambda i: (i, 0)),
        ],
        out_specs=[
            pl.BlockSpec((IMGB * CPAD, HW), lambda i: (i, 0)),
            pl.BlockSpec((2 * CPAD, 128), lambda i: (0, 0)),
        ],
        out_shape=[
            jax.ShapeDtypeStruct((N * CPAD, HW), jnp.bfloat16),
            jax.ShapeDtypeStruct((2 * CPAD, 128), jnp.float32),
        ],
        scratch_shapes=[
            pltpu.VMEM((rows, HW + 2 * PAD), jnp.bfloat16),
            pltpu.VMEM((9 * CPAD + 8, IMGB * HW), jnp.bfloat16),
        ],
        compiler_params=pltpu.CompilerParams(
            dimension_semantics=("arbitrary",),
            vmem_limit_bytes=_VMEM_LIMIT),
    )(aux0, w0b, xb)
    return h, acc


IMGB1 = 8


def _conv1_kernel(aux_ref, w_ref, h_ref, x_ref, out_ref, apad_ref, patch_ref,
                  *, C, W, HW):
    i = pl.program_id(0)
    fout, H = out_ref.shape[1], out_ref.shape[2]

    @pl.when(i == 0)
    def _():
        apad_ref[...] = jnp.zeros_like(apad_ref)
        patch_ref[...] = jnp.zeros_like(patch_ref)
        ones = patch_ref[10 * CPAD:, :]
        patch_ref[10 * CPAD:, :] = jnp.ones_like(ones)

    a = _leaky_bn(h_ref[...].astype(jnp.float32), aux_ref).astype(jnp.bfloat16)
    apad_ref[:, PAD:PAD + HW] = a
    masks = _col_masks(C, W, HW)
    for b in range(IMGB1):
        _fill_taps(apad_ref, patch_ref, b, CPAD, C, W, HW, masks)
        patch_ref[9 * CPAD:9 * CPAD + C, b * HW:(b + 1) * HW] = (
            x_ref[b * C:(b + 1) * C, :])
    out = jnp.dot(w_ref[...], patch_ref[...],
                  preferred_element_type=jnp.float32)
    for b in range(IMGB1):
        out_ref[b] = out[0:fout, b * HW:(b + 1) * HW].reshape(
            fout, out_ref.shape[2], out_ref.shape[3])


def _conv1(h2, xb, aux1, w1b, C, fout, H, W):
    HW = H * W
    R = xb.shape[0]
    N = R // C
    rows = IMGB1 * C
    HR = HW // 128
    return pl.pallas_call(
        functools.partial(_conv1_kernel, C=C, W=W, HW=HW),
        grid=(R // rows,),
        in_specs=[
            pl.BlockSpec(aux1.shape, lambda i: (0, 0)),
            pl.BlockSpec(w1b.shape, lambda i: (0, 0)),
            pl.BlockSpec((IMGB1 * CPAD, HW), lambda i: (i, 0)),
            pl.BlockSpec((rows, HW), lambda i: (i, 0)),
        ],
        out_specs=pl.BlockSpec((IMGB1, fout, HR, 128), lambda i: (i, 0, 0, 0)),
        out_shape=jax.ShapeDtypeStruct((N, fout, HR, 128), jnp.float32),
        scratch_shapes=[
            pltpu.VMEM((IMGB1 * CPAD, HW + 2 * PAD), jnp.bfloat16),
            pltpu.VMEM((10 * CPAD + 8, IMGB1 * HW), jnp.bfloat16),
        ],
        compiler_params=pltpu.CompilerParams(
            dimension_semantics=("arbitrary",),
            vmem_limit_bytes=_VMEM_LIMIT),
    )(aux1, w1b, h2, xb)


def kernel(x_nchw, w0, b0, w1, b1, bn0_g, bn0_b, bn1_g, bn1_b):
    N, C, H, W = x_nchw.shape
    HW = H * W
    fout = b1.shape[1]

    w0t = jnp.transpose(w0)[0:CPAD, :]
    w1t = jnp.transpose(w1)[0:CPAD, :]
    w0b = jnp.concatenate(
        [_retile_taps(w0t, C), _bias_band(b0)],
        axis=1).astype(jnp.bfloat16)
    w1b = jnp.concatenate(
        [_retile_taps(w1t, C), w1t[:, 9 * C:9 * C + C],
         jnp.zeros((CPAD, CPAD - C), jnp.float32), _bias_band(b1)],
        axis=1).astype(jnp.bfloat16)

    xb, mean0, inv_std0 = _channel_stats(
        x_nchw.reshape(N, C, HW // 128, 128))
    aux0 = _make_aux(mean0, inv_std0, bn0_g, bn0_b, C, C, IMGB)

    h2, acc1 = _conv0(xb, aux0, w0b, C, W, HW)

    count = float(N * HW)
    s1 = jnp.sum(acc1[0:C, :], axis=1)
    ss1 = jnp.sum(acc1[CPAD:CPAD + C, :], axis=1)
    mean1 = s1 / count
    var1 = jnp.maximum(ss1 / count - mean1 * mean1, 0.0)
    inv_std1 = 1.0 / jnp.sqrt(var1 + BN_EPS)
    aux1 = _make_aux(mean1, inv_std1, bn1_g, bn1_b, CPAD, C, IMGB1)

    out32 = _conv1(h2, xb, aux1, w1b, C, fout, H, W)
    return out32.reshape(N, fout, H, W)
